# baseline (speedup 1.0000x reference)
import functools

import jax
import jax.numpy as jnp
from jax import lax
from jax.experimental import pallas as pl
from jax.experimental.pallas import tpu as pltpu
from jax.experimental.pallas import tpu_sc as plsc

D_MODEL = 64
NC, NS = 2, 16
NW = NC * NS
BBLK = 128
PADR = 73
SCALE = 8.0


def _positional_encoding(length, d_model):
    depth = d_model / 2
    pos = jnp.arange(0, length, dtype=jnp.float32)[:, None]
    i = jnp.arange(0, depth, dtype=jnp.float32)
    angle = pos / jnp.power(10000.0, 2.0 * i / depth)
    return jnp.concatenate([jnp.sin(angle), jnp.cos(angle)], axis=-1)


def _sc_embed(xt, W, pe, B, L):
    assert B == NW * BBLK
    n_iter = L // 2
    mesh = plsc.VectorSubcoreMesh(core_axis_name="c", subcore_axis_name="s")

    @functools.partial(
        pl.kernel,
        mesh=mesh,
        out_type=jax.ShapeDtypeStruct(
            (L, D_MODEL // 8, B // BBLK, 8, BBLK), jnp.float32),
        scratch_types=[
            pltpu.VMEM((L, BBLK), jnp.int32),
            pltpu.VMEM((BBLK, D_MODEL), jnp.float32),
            pltpu.VMEM((BBLK, D_MODEL), jnp.float32),
            pltpu.VMEM((BBLK * PADR,), jnp.float32),
            pltpu.VMEM((D_MODEL // 8, 8, BBLK), jnp.float32),
            pltpu.VMEM((D_MODEL // 8, 8, BBLK), jnp.float32),
            pltpu.VMEM((L, D_MODEL), jnp.float32),
            pltpu.SemaphoreType.DMA,
            pltpu.SemaphoreType.DMA,
            pltpu.SemaphoreType.DMA,
            pltpu.SemaphoreType.DMA,
            pltpu.SemaphoreType.DMA,
        ],
        compiler_params=pltpu.CompilerParams(
            use_tc_tiling_on_sc=False, needs_layout_passes=False),
    )
    def k(xt_hbm, w_hbm, pe_hbm, out_hbm, idx_all, rows0, rows1, rpad,
          comp0, comp1, pe_v, isem, gsem0, gsem1, osem0, osem1):
        wid = lax.axis_index("s") * NC + lax.axis_index("c")
        pltpu.async_copy(
            xt_hbm.at[pl.ds(0, L), pl.ds(wid * BBLK, BBLK)], idx_all, isem)
        pltpu.sync_copy(pe_hbm, pe_v)
        pltpu.make_async_copy(
            xt_hbm.at[pl.ds(0, L), pl.ds(wid * BBLK, BBLK)], idx_all,
            isem).wait()

        rows = (rows0, rows1)
        comp = (comp0, comp1)
        gsem = (gsem0, gsem1)
        osem = (osem0, osem1)

        lane = jnp.arange(16, dtype=jnp.int32)
        l_pad = lane * PADR

        def start_gather(s, b):
            pltpu.async_copy(w_hbm.at[idx_all.at[s]], rows[b], gsem[b])

        def wait_gather(s, b):
            pltpu.make_async_copy(
                w_hbm.at[idx_all.at[s]], rows[b], gsem[b]).wait()

        start_gather(0, 0)
        start_gather(1, 1)

        def iter_body(i, carry):
            for b in range(2):
                s = 2 * i + b
                wait_gather(s, b)

                def _wait_out(bb=b, ss=s):
                    pltpu.make_async_copy(
                        comp[bb], out_hbm.at[ss - 2, pl.ds(0, D_MODEL // 8),
                                             wid], osem[bb]).wait()

                pl.when(i > 0)(_wait_out)

                pe16s = [pe_v[s, pl.ds(j * 16, 16)] for j in range(4)]

                def p1_body(cc, off, bb=b, pp=pe16s):
                    for j in range(D_MODEL // 16):
                        v = rows[bb][cc, pl.ds(j * 16, 16)] * SCALE + pp[j]
                        rpad[pl.ds(off + j * 16, 16)] = v
                    return off + PADR

                plsc.parallel_loop(
                    0, BBLK, unroll=4, carry=jnp.int32(0))(p1_body)

                def p2_body(d, doff, bb=b):
                    tr = lax.shift_right_logical(d, 3)
                    r = lax.bitwise_and(d, 7)
                    for c0 in range(BBLK // 16):
                        idx = l_pad + (c0 * 16 * PADR) + doff
                        v = plsc.load_gather(rpad, [idx])
                        comp[bb][tr, r, pl.ds(c0 * 16, 16)] = v
                    return doff + 1

                plsc.parallel_loop(
                    0, D_MODEL, unroll=4, carry=jnp.int32(0))(p2_body)

                pltpu.async_copy(
                    comp[b], out_hbm.at[s, pl.ds(0, D_MODEL // 8), wid],
                    osem[b])

                def _next_gather(bb=b, ss=s):
                    start_gather(ss + 2, bb)

                pl.when(i < n_iter - 1)(_next_gather)
            return carry

        lax.fori_loop(0, n_iter, iter_body, 0)
        pltpu.make_async_copy(
            comp0, out_hbm.at[L - 2, pl.ds(0, D_MODEL // 8), wid],
            osem0).wait()
        pltpu.make_async_copy(
            comp1, out_hbm.at[L - 1, pl.ds(0, D_MODEL // 8), wid],
            osem1).wait()

    return k(xt, W, pe)


def kernel(x, W):
    B, L = x.shape
    xt = x.T
    pe = _positional_encoding(L, D_MODEL)
    out5 = _sc_embed(xt, W, pe, B, L)
    return out5.transpose(2, 4, 0, 1, 3).reshape(B, L, D_MODEL)


# speedup vs baseline: 1.1406x; 1.1406x over previous
import functools

import jax
import jax.numpy as jnp
from jax import lax
from jax.experimental import pallas as pl
from jax.experimental.pallas import tpu as pltpu
from jax.experimental.pallas import tpu_sc as plsc

D_MODEL = 64
NC, NS = 2, 16
NW = NC * NS
BBLK = 128
PADR = 73
SCALE = 8.0


def _positional_encoding(length, d_model):
    depth = d_model / 2
    pos = jnp.arange(0, length, dtype=jnp.float32)[:, None]
    i = jnp.arange(0, depth, dtype=jnp.float32)
    angle = pos / jnp.power(10000.0, 2.0 * i / depth)
    return jnp.concatenate([jnp.sin(angle), jnp.cos(angle)], axis=-1)


def _sc_embed(xt, W, pe, B, L):
    assert B == NW * BBLK
    n_iter = L // 4
    mesh = plsc.VectorSubcoreMesh(core_axis_name="c", subcore_axis_name="s")

    @functools.partial(
        pl.kernel,
        mesh=mesh,
        out_type=jax.ShapeDtypeStruct(
            (L, D_MODEL // 8, B // BBLK, 8, BBLK), jnp.float32),
        scratch_types=[
            pltpu.VMEM((L, BBLK), jnp.int32),
            pltpu.VMEM((2 * BBLK, D_MODEL), jnp.float32),
            pltpu.VMEM((2 * BBLK, D_MODEL), jnp.float32),
            pltpu.VMEM((BBLK * PADR,), jnp.float32),
            pltpu.VMEM((2, D_MODEL // 8, 8, BBLK), jnp.float32),
            pltpu.VMEM((2, D_MODEL // 8, 8, BBLK), jnp.float32),
            pltpu.VMEM((L, D_MODEL), jnp.float32),
            pltpu.SemaphoreType.DMA,
            pltpu.SemaphoreType.DMA,
            pltpu.SemaphoreType.DMA,
            pltpu.SemaphoreType.DMA,
            pltpu.SemaphoreType.DMA,
        ],
        compiler_params=pltpu.CompilerParams(
            use_tc_tiling_on_sc=False, needs_layout_passes=False),
    )
    def k(xt_hbm, w_hbm, pe_hbm, out_hbm, idx_all, rows0, rows1, rpad,
          comp0, comp1, pe_v, isem, gsem0, gsem1, osem0, osem1):
        wid = lax.axis_index("s") * NC + lax.axis_index("c")
        pltpu.async_copy(
            xt_hbm.at[pl.ds(0, L), pl.ds(wid * BBLK, BBLK)], idx_all, isem)
        pltpu.sync_copy(pe_hbm, pe_v)
        pltpu.make_async_copy(
            xt_hbm.at[pl.ds(0, L), pl.ds(wid * BBLK, BBLK)], idx_all,
            isem).wait()

        rows = (rows0, rows1)
        comp = (comp0, comp1)
        gsem = (gsem0, gsem1)
        osem = (osem0, osem1)

        lane = jnp.arange(16, dtype=jnp.int32)
        l_pad = lane * PADR

        def start_gather(step, b):
            for p in range(2):
                pltpu.async_copy(
                    w_hbm.at[idx_all.at[2 * step + p]],
                    rows[b].at[pl.ds(p * BBLK, BBLK)], gsem[b])

        def wait_gather(step, b):
            for p in range(2):
                pltpu.make_async_copy(
                    w_hbm.at[idx_all.at[2 * step + p]],
                    rows[b].at[pl.ds(p * BBLK, BBLK)], gsem[b]).wait()

        start_gather(0, 0)
        start_gather(1, 1)

        def iter_body(i, carry):
            for b in range(2):
                step = 2 * i + b
                s0 = 2 * step
                wait_gather(step, b)

                def _wait_out(bb=b, ss=s0):
                    pltpu.make_async_copy(
                        comp[bb],
                        out_hbm.at[pl.ds(ss - 4, 2), pl.ds(0, D_MODEL // 8),
                                   wid], osem[bb]).wait()

                pl.when(i > 0)(_wait_out)

                for p in range(2):
                    pe16s = [pe_v[s0 + p, pl.ds(j * 16, 16)] for j in range(4)]

                    def p1_body(cc, off, bb=b, pp=pe16s, base=p * BBLK):
                        for j in range(D_MODEL // 16):
                            v = (rows[bb][base + cc, pl.ds(j * 16, 16)] * SCALE
                                 + pp[j])
                            rpad[pl.ds(off + j * 16, 16)] = v
                        return off + PADR

                    plsc.parallel_loop(
                        0, BBLK, unroll=4, carry=jnp.int32(0))(p1_body)

                    def p2_body(d, doff, bb=b, pp=p):
                        tr = lax.shift_right_logical(d, 3)
                        r = lax.bitwise_and(d, 7)
                        for c0 in range(BBLK // 16):
                            idx = l_pad + (c0 * 16 * PADR) + doff
                            v = plsc.load_gather(rpad, [idx])
                            comp[bb][pp, tr, r, pl.ds(c0 * 16, 16)] = v
                        return doff + 1

                    plsc.parallel_loop(
                        0, D_MODEL, unroll=4, carry=jnp.int32(0))(p2_body)

                pltpu.async_copy(
                    comp[b],
                    out_hbm.at[pl.ds(s0, 2), pl.ds(0, D_MODEL // 8), wid],
                    osem[b])

                def _next_gather(bb=b, st=step):
                    start_gather(st + 2, bb)

                pl.when(i < n_iter - 1)(_next_gather)
            return carry

        lax.fori_loop(0, n_iter, iter_body, 0)
        pltpu.make_async_copy(
            comp0, out_hbm.at[pl.ds(L - 4, 2), pl.ds(0, D_MODEL // 8), wid],
            osem0).wait()
        pltpu.make_async_copy(
            comp1, out_hbm.at[pl.ds(L - 2, 2), pl.ds(0, D_MODEL // 8), wid],
            osem1).wait()

    return k(xt, W, pe)


def kernel(x, W):
    B, L = x.shape
    xt = x.T
    pe = _positional_encoding(L, D_MODEL)
    out5 = _sc_embed(xt, W, pe, B, L)
    return out5.transpose(2, 4, 0, 1, 3).reshape(B, L, D_MODEL)


# speedup vs baseline: 1.1414x; 1.0007x over previous
import functools

import jax
import jax.numpy as jnp
from jax import lax
from jax.experimental import pallas as pl
from jax.experimental.pallas import tpu as pltpu
from jax.experimental.pallas import tpu_sc as plsc

D_MODEL = 64
NC, NS = 2, 16
NW = NC * NS
BBLK = 128
PADR = 73
SCALE = 8.0


def _positional_encoding(length, d_model):
    depth = d_model / 2
    pos = jnp.arange(0, length, dtype=jnp.float32)[:, None]
    i = jnp.arange(0, depth, dtype=jnp.float32)
    angle = pos / jnp.power(10000.0, 2.0 * i / depth)
    return jnp.concatenate([jnp.sin(angle), jnp.cos(angle)], axis=-1)


def _sc_embed(xt, W, pe, B, L):
    assert B == NW * BBLK
    n_iter = L // 4
    mesh = plsc.VectorSubcoreMesh(core_axis_name="c", subcore_axis_name="s")

    @functools.partial(
        pl.kernel,
        mesh=mesh,
        out_type=jax.ShapeDtypeStruct(
            (L, D_MODEL // 8, B // BBLK, 8, BBLK), jnp.float32),
        scratch_types=[
            pltpu.VMEM((L, BBLK), jnp.int32),
            pltpu.VMEM((2 * BBLK, D_MODEL), jnp.float32),
            pltpu.VMEM((2 * BBLK, D_MODEL), jnp.float32),
            pltpu.VMEM((BBLK * PADR,), jnp.float32),
            pltpu.VMEM((2, D_MODEL // 8, 8, BBLK), jnp.float32),
            pltpu.VMEM((2, D_MODEL // 8, 8, BBLK), jnp.float32),
            pltpu.VMEM((L, D_MODEL), jnp.float32),
            pltpu.SemaphoreType.DMA,
            pltpu.SemaphoreType.DMA,
            pltpu.SemaphoreType.DMA,
            pltpu.SemaphoreType.DMA,
            pltpu.SemaphoreType.DMA,
        ],
        compiler_params=pltpu.CompilerParams(
            use_tc_tiling_on_sc=False, needs_layout_passes=False),
    )
    def k(xt_hbm, w_hbm, pe_hbm, out_hbm, idx_all, rows0, rows1, rpad,
          comp0, comp1, pe_v, isem, gsem0, gsem1, osem0, osem1):
        wid = lax.axis_index("s") * NC + lax.axis_index("c")
        pltpu.async_copy(
            xt_hbm.at[pl.ds(0, L), pl.ds(wid * BBLK, BBLK)], idx_all, isem)
        pltpu.sync_copy(pe_hbm, pe_v)
        pltpu.make_async_copy(
            xt_hbm.at[pl.ds(0, L), pl.ds(wid * BBLK, BBLK)], idx_all,
            isem).wait()

        rows = (rows0, rows1)
        comp = (comp0, comp1)
        gsem = (gsem0, gsem1)
        osem = (osem0, osem1)

        lane = jnp.arange(16, dtype=jnp.int32)
        l_pad = lane * PADR

        def start_gather(step, b):
            for p in range(2):
                pltpu.async_copy(
                    w_hbm.at[idx_all.at[2 * step + p]],
                    rows[b].at[pl.ds(p * BBLK, BBLK)], gsem[b])

        def wait_gather(step, b):
            for p in range(2):
                pltpu.make_async_copy(
                    w_hbm.at[idx_all.at[2 * step + p]],
                    rows[b].at[pl.ds(p * BBLK, BBLK)], gsem[b]).wait()

        start_gather(0, 0)
        start_gather(1, 1)

        def iter_body(i, carry):
            for b in range(2):
                step = 2 * i + b
                s0 = 2 * step
                wait_gather(step, b)

                def _wait_out(bb=b, ss=s0):
                    pltpu.make_async_copy(
                        comp[bb],
                        out_hbm.at[pl.ds(ss - 4, 2), pl.ds(0, D_MODEL // 8),
                                   wid], osem[bb]).wait()

                pl.when(i > 0)(_wait_out)

                for p in range(2):
                    pe16s = [pe_v[s0 + p, pl.ds(j * 16, 16)] for j in range(4)]

                    def p1_body(cc, off, bb=b, pp=pe16s, base=p * BBLK):
                        for j in range(D_MODEL // 16):
                            v = (rows[bb][base + cc, pl.ds(j * 16, 16)] * SCALE
                                 + pp[j])
                            rpad[pl.ds(off + j * 16, 16)] = v
                        return off + PADR

                    plsc.parallel_loop(
                        0, BBLK, unroll=8, carry=jnp.int32(0))(p1_body)

                    def p2_body(d, doff, bb=b, pp=p):
                        tr = lax.shift_right_logical(d, 3)
                        r = lax.bitwise_and(d, 7)
                        for c0 in range(BBLK // 16):
                            idx = l_pad + (c0 * 16 * PADR) + doff
                            v = plsc.load_gather(rpad, [idx])
                            comp[bb][pp, tr, r, pl.ds(c0 * 16, 16)] = v
                        return doff + 1

                    plsc.parallel_loop(
                        0, D_MODEL, unroll=8, carry=jnp.int32(0))(p2_body)

                pltpu.async_copy(
                    comp[b],
                    out_hbm.at[pl.ds(s0, 2), pl.ds(0, D_MODEL // 8), wid],
                    osem[b])

                def _next_gather(bb=b, st=step):
                    start_gather(st + 2, bb)

                pl.when(i < n_iter - 1)(_next_gather)
            return carry

        lax.fori_loop(0, n_iter, iter_body, 0)
        pltpu.make_async_copy(
            comp0, out_hbm.at[pl.ds(L - 4, 2), pl.ds(0, D_MODEL // 8), wid],
            osem0).wait()
        pltpu.make_async_copy(
            comp1, out_hbm.at[pl.ds(L - 2, 2), pl.ds(0, D_MODEL // 8), wid],
            osem1).wait()

    return k(xt, W, pe)


def kernel(x, W):
    B, L = x.shape
    xt = x.T
    pe = _positional_encoding(L, D_MODEL)
    out5 = _sc_embed(xt, W, pe, B, L)
    return out5.transpose(2, 4, 0, 1, 3).reshape(B, L, D_MODEL)


# speedup vs baseline: 1.1944x; 1.0464x over previous
import functools

import jax
import jax.numpy as jnp
from jax import lax
from jax.experimental import pallas as pl
from jax.experimental.pallas import tpu as pltpu
from jax.experimental.pallas import tpu_sc as plsc

D_MODEL = 64
NC, NS = 2, 16
NW = NC * NS
BBLK = 128
PADR = 73
SCALE = 8.0


def _positional_encoding(length, d_model):
    depth = d_model / 2
    pos = jnp.arange(0, length, dtype=jnp.float32)[:, None]
    i = jnp.arange(0, depth, dtype=jnp.float32)
    angle = pos / jnp.power(10000.0, 2.0 * i / depth)
    return jnp.concatenate([jnp.sin(angle), jnp.cos(angle)], axis=-1)


def _sc_embed(xt, W, pe, B, L):
    assert B == NW * BBLK
    n_iter = L // 4
    mesh = plsc.VectorSubcoreMesh(core_axis_name="c", subcore_axis_name="s")

    @functools.partial(
        pl.kernel,
        mesh=mesh,
        out_type=jax.ShapeDtypeStruct(
            (L, D_MODEL // 8, B // BBLK, 8, BBLK), jnp.float32),
        scratch_types=[
            pltpu.VMEM((L, BBLK), jnp.int32),
            pltpu.VMEM((2 * BBLK, D_MODEL), jnp.float32),
            pltpu.VMEM((2 * BBLK, D_MODEL), jnp.float32),
            pltpu.VMEM((BBLK * PADR,), jnp.float32),
            pltpu.VMEM((2, D_MODEL // 8, 8, BBLK), jnp.float32),
            pltpu.VMEM((2, D_MODEL // 8, 8, BBLK), jnp.float32),
            pltpu.VMEM((L, D_MODEL), jnp.float32),
            pltpu.SemaphoreType.DMA,
            pltpu.SemaphoreType.DMA,
            pltpu.SemaphoreType.DMA,
            pltpu.SemaphoreType.DMA,
            pltpu.SemaphoreType.DMA,
        ],
        compiler_params=pltpu.CompilerParams(
            use_tc_tiling_on_sc=False, needs_layout_passes=False),
    )
    def k(xt_hbm, w_hbm, pe_hbm, out_hbm, idx_all, rows0, rows1, rpad,
          comp0, comp1, pe_v, isem, gsem0, gsem1, osem0, osem1):
        wid = lax.axis_index("s") * NC + lax.axis_index("c")
        pltpu.async_copy(
            xt_hbm.at[pl.ds(0, L), pl.ds(wid * BBLK, BBLK)], idx_all, isem)
        pltpu.sync_copy(pe_hbm, pe_v)
        pltpu.make_async_copy(
            xt_hbm.at[pl.ds(0, L), pl.ds(wid * BBLK, BBLK)], idx_all,
            isem).wait()

        rows = (rows0, rows1)
        comp = (comp0, comp1)
        gsem = (gsem0, gsem1)
        osem = (osem0, osem1)

        lane = jnp.arange(16, dtype=jnp.int32)
        l_pad = lane * PADR

        def start_gather(step, b):
            for p in range(2):
                pltpu.async_copy(
                    w_hbm.at[idx_all.at[2 * step + p]],
                    rows[b].at[pl.ds(p * BBLK, BBLK)], gsem[b])

        def wait_gather(step, b):
            for p in range(2):
                pltpu.make_async_copy(
                    w_hbm.at[idx_all.at[2 * step + p]],
                    rows[b].at[pl.ds(p * BBLK, BBLK)], gsem[b]).wait()

        start_gather(0, 0)
        start_gather(1, 1)

        def iter_body(i, carry):
            for b in range(2):
                step = 2 * i + b
                s0 = 2 * step
                wait_gather(step, b)

                def _wait_out(bb=b, ss=s0):
                    pltpu.make_async_copy(
                        comp[bb],
                        out_hbm.at[pl.ds(ss - 4, 2), pl.ds(0, D_MODEL // 8),
                                   wid], osem[bb]).wait()

                pl.when(i > 0)(_wait_out)

                for p in range(2):
                    pe16s = [pe_v[s0 + p, pl.ds(j * 16, 16)] for j in range(4)]

                    def p1_body(cc, off, bb=b, pp=pe16s, base=p * BBLK):
                        for j in range(D_MODEL // 16):
                            v = (rows[bb][base + cc, pl.ds(j * 16, 16)] * SCALE
                                 + pp[j])
                            rpad[pl.ds(off + j * 16, 16)] = v
                        return off + PADR

                    plsc.parallel_loop(
                        0, BBLK, unroll=8, carry=jnp.int32(0))(p1_body)

                    def _next_gather_half(bb=b, st=step, ph=p):
                        pltpu.async_copy(
                            w_hbm.at[idx_all.at[2 * (st + 2) + ph]],
                            rows[bb].at[pl.ds(ph * BBLK, BBLK)], gsem[bb])

                    pl.when(i < n_iter - 1)(_next_gather_half)

                    def p2_body(d, doff, bb=b, pp=p):
                        tr = lax.shift_right_logical(d, 3)
                        r = lax.bitwise_and(d, 7)
                        for c0 in range(BBLK // 16):
                            idx = l_pad + (c0 * 16 * PADR) + doff
                            v = plsc.load_gather(rpad, [idx])
                            comp[bb][pp, tr, r, pl.ds(c0 * 16, 16)] = v
                        return doff + 1

                    plsc.parallel_loop(
                        0, D_MODEL, unroll=8, carry=jnp.int32(0))(p2_body)

                pltpu.async_copy(
                    comp[b],
                    out_hbm.at[pl.ds(s0, 2), pl.ds(0, D_MODEL // 8), wid],
                    osem[b])
            return carry

        lax.fori_loop(0, n_iter, iter_body, 0)
        pltpu.make_async_copy(
            comp0, out_hbm.at[pl.ds(L - 4, 2), pl.ds(0, D_MODEL // 8), wid],
            osem0).wait()
        pltpu.make_async_copy(
            comp1, out_hbm.at[pl.ds(L - 2, 2), pl.ds(0, D_MODEL // 8), wid],
            osem1).wait()

    return k(xt, W, pe)


def kernel(x, W):
    B, L = x.shape
    xt = x.T
    pe = _positional_encoding(L, D_MODEL)
    out5 = _sc_embed(xt, W, pe, B, L)
    return out5.transpose(2, 4, 0, 1, 3).reshape(B, L, D_MODEL)
